# baseline (speedup 1.0000x reference)
import functools

import jax
import jax.numpy as jnp
from jax import lax
from jax.experimental import pallas as pl
from jax.experimental.pallas import tpu as pltpu
from jax.experimental.pallas import tpu_sc as plsc

_N = 10000
_E = 320000
_NSC = 2
_NTILE = 16
_NW = _NSC * _NTILE
_RPT = 624
_TAIL = _N - _NTILE * _RPT
_C = 80
_NCH = 128
_NCHP = _NCH + 4
_EWP = _NCHP * _C
_UNROLL = 8

_NB = 4
_NRB = 8


def _spmm_body(D, m_hbm, col_hbm, row_hbm, w_hbm, out_hbm,
               colb, wb, rowb, rows0, rows1, rows2, rows3, acc,
               semg, sems, semi):
    c = lax.axis_index("c")
    s = lax.axis_index("s")
    wid = c * _NTILE + s
    rows = (rows0, rows1, rows2, rows3)

    def zrow(i, carry):
        for j in range(D // 16):
            rows3[i, pl.ds(j * 16, 16)] = jnp.zeros((16,), jnp.float32)
        return carry

    lax.fori_loop(0, _C, zrow, 0)
    base = s * _RPT
    for k in range(_RPT // _C):
        pltpu.sync_copy(rows3, acc.at[pl.ds(base + k * _C, _C)])
    _rem = _RPT - (_RPT // _C) * _C
    pltpu.sync_copy(rows3.at[pl.ds(0, _rem)],
                    acc.at[pl.ds(base + (_RPT // _C) * _C, _rem)])

    @pl.when(s == _NTILE - 1)
    def _zero_tail():
        pltpu.sync_copy(rows3.at[pl.ds(0, _TAIL)],
                        acc.at[pl.ds(_NTILE * _RPT, _TAIL)])

    def issue_idx(k, i8):
        b = i8 % _NB
        pltpu.async_copy(col_hbm.at[wid, k], colb.at[b], semi.at[i8])
        pltpu.async_copy(row_hbm.at[wid, k], rowb.at[i8], semi.at[i8])
        pltpu.async_copy(w_hbm.at[wid, k], wb.at[b], semi.at[i8])

    def wait_idx(i8):
        b = i8 % _NB
        pltpu.make_async_copy(col_hbm.at[wid, 0], colb.at[b],
                              semi.at[i8]).wait()
        pltpu.make_async_copy(row_hbm.at[wid, 0], rowb.at[i8],
                              semi.at[i8]).wait()
        pltpu.make_async_copy(w_hbm.at[wid, 0], wb.at[b],
                              semi.at[i8]).wait()

    def issue_gather(b):
        pltpu.async_copy(m_hbm.at[colb.at[b]], rows[b], semg.at[b])

    def wait_gather(b):
        pltpu.make_async_copy(m_hbm.at[colb.at[b]], rows[b],
                              semg.at[b]).wait()

    def issue_scatter(b, i8):
        pltpu.async_copy(rows[b], acc.at[rowb.at[i8]], sems.at[b],
                         add=True)

    def wait_scatter(b, i8):
        pltpu.make_async_copy(rows[b], acc.at[rowb.at[i8]],
                              sems.at[b]).wait()

    for u in range(4):
        issue_idx(u, u)
    wait_idx(0)
    issue_gather(0)
    wait_idx(1)
    issue_gather(1)
    plsc.subcore_barrier()

    def scale(b, i8):
        buf = rows[b]

        def body(g, inner):
            w16 = wb[i8 % _NB, pl.ds(g * 16, 16)]
            for l in range(16):
                e = g * 16 + l
                wspl = jnp.full((16,), w16[l], jnp.float32)
                for j in range(D // 16):
                    buf[e, pl.ds(j * 16, 16)] = (
                        buf[e, pl.ds(j * 16, 16)] * wspl)
            return inner

        lax.fori_loop(0, _C // 16, body, 0)

    def octet(j, carry):
        k8 = j * _UNROLL
        for u in range(_UNROLL):
            b = u % _NB
            wait_gather(b)
            scale(b, u)
            issue_scatter(b, u)
            issue_idx(k8 + u + 4, (u + 4) % _NRB)
            wait_idx((u + 2) % _NRB)
            if u >= 2:
                wait_scatter((u + 2) % _NB, (u - 2) % _NRB)
            else:
                @pl.when(j > 0)
                def _ws():
                    wait_scatter((u + 2) % _NB, (u - 2) % _NRB)
            issue_gather((u + 2) % _NB)
        return carry

    lax.fori_loop(0, _NCH // _UNROLL, octet, 0)
    wait_gather(0)
    wait_gather(1)
    wait_scatter(2, 6)
    wait_scatter(3, 7)
    wait_idx(2)
    wait_idx(3)
    plsc.subcore_barrier()

    pltpu.sync_copy(acc.at[pl.ds(s * _RPT, _RPT)],
                    out_hbm.at[c, pl.ds(s * _RPT, _RPT)])

    @pl.when(s == _NTILE - 1)
    def _write_tail():
        pltpu.sync_copy(acc.at[pl.ds(_NTILE * _RPT, _TAIL)],
                        out_hbm.at[c, pl.ds(_NTILE * _RPT, _TAIL)])


@functools.cache
def _make_spmm(D):
    mesh = plsc.VectorSubcoreMesh(core_axis_name="c", subcore_axis_name="s")
    return pl.kernel(
        functools.partial(_spmm_body, D),
        out_type=jax.ShapeDtypeStruct((_NSC, _N, D), jnp.float32),
        mesh=mesh,
        scratch_types=[
            pltpu.VMEM((_NB, _C), jnp.int32),
            pltpu.VMEM((_NB, _C), jnp.float32),
            pltpu.VMEM((_NRB, _C), jnp.int32),
            pltpu.VMEM((_C, D), jnp.float32),
            pltpu.VMEM((_C, D), jnp.float32),
            pltpu.VMEM((_C, D), jnp.float32),
            pltpu.VMEM((_C, D), jnp.float32),
            pltpu.VMEM_SHARED((_N, D), jnp.float32),
            pltpu.SemaphoreType.DMA((_NB,)),
            pltpu.SemaphoreType.DMA((_NB,)),
            pltpu.SemaphoreType.DMA((_NRB,)),
        ],
        name=f"gcn_spmm_d{D}",
    )


def _matmul_body(x_ref, w_ref, o_ref):
    o_ref[...] = jnp.dot(x_ref[...], w_ref[...],
                         preferred_element_type=jnp.float32)


def _fused_body(p0_ref, p1_ref, b_ref, w_ref, o_ref):
    h = jnp.maximum(p0_ref[...] + p1_ref[...] + b_ref[...], 0.0)
    o_ref[...] = jnp.dot(h, w_ref[...], preferred_element_type=jnp.float32)


def _combine_relu_body(p0_ref, p1_ref, b_ref, o_ref):
    o_ref[...] = jnp.maximum(p0_ref[...] + p1_ref[...] + b_ref[...], 0.0)


def _final_body(p0_ref, p1_ref, w_ref, b_ref, o_ref):
    o_ref[...] = jnp.dot(p0_ref[...] + p1_ref[...], w_ref[...],
                         preferred_element_type=jnp.float32) + b_ref[...]


_BLK = 1000


def _matmul(x, W):
    K, M = W.shape
    return pl.pallas_call(
        _matmul_body,
        grid=(_N // _BLK,),
        in_specs=[
            pl.BlockSpec((_BLK, K), lambda i: (i, 0)),
            pl.BlockSpec((K, M), lambda i: (0, 0)),
        ],
        out_specs=pl.BlockSpec((_BLK, M), lambda i: (i, 0)),
        out_shape=jax.ShapeDtypeStruct((_N, M), jnp.float32),
    )(x, W)


def _fused(p0, p1, b, W):
    K, M = W.shape
    return pl.pallas_call(
        _fused_body,
        grid=(_N // _BLK,),
        in_specs=[
            pl.BlockSpec((_BLK, K), lambda i: (i, 0)),
            pl.BlockSpec((_BLK, K), lambda i: (i, 0)),
            pl.BlockSpec((1, K), lambda i: (0, 0)),
            pl.BlockSpec((K, M), lambda i: (0, 0)),
        ],
        out_specs=pl.BlockSpec((_BLK, M), lambda i: (i, 0)),
        out_shape=jax.ShapeDtypeStruct((_N, M), jnp.float32),
    )(p0, p1, b.reshape(1, K), W)


def _combine_relu(p0, p1, b):
    M = p0.shape[1]
    return pl.pallas_call(
        _combine_relu_body,
        grid=(_N // _BLK,),
        in_specs=[
            pl.BlockSpec((_BLK, M), lambda i: (i, 0)),
            pl.BlockSpec((_BLK, M), lambda i: (i, 0)),
            pl.BlockSpec((1, M), lambda i: (0, 0)),
        ],
        out_specs=pl.BlockSpec((_BLK, M), lambda i: (i, 0)),
        out_shape=jax.ShapeDtypeStruct((_N, M), jnp.float32),
    )(p0, p1, b.reshape(1, M))


def _final(p0, p1, W, b):
    K, M = W.shape
    return pl.pallas_call(
        _final_body,
        grid=(_N // _BLK,),
        in_specs=[
            pl.BlockSpec((_BLK, K), lambda i: (i, 0)),
            pl.BlockSpec((_BLK, K), lambda i: (i, 0)),
            pl.BlockSpec((K, M), lambda i: (0, 0)),
            pl.BlockSpec((1, M), lambda i: (0, 0)),
        ],
        out_specs=pl.BlockSpec((_BLK, M), lambda i: (i, 0)),
        out_shape=jax.ShapeDtypeStruct((_N, M), jnp.float32),
    )(p0, p1, W, b.reshape(1, M))


def kernel(x, edge_index, edge_weight, W0, b0, W1, b1, W2, b2):
    ew_proc = _NCH * _C
    pad = _NW * ew_proc - _E

    def _prep(a):
        a2 = jnp.pad(a, (0, pad)).reshape(_NW, ew_proc)
        a2 = jnp.pad(a2, ((0, 0), (0, _EWP - ew_proc)))
        return a2.reshape(_NW, _NCHP, _C)

    row = _prep(edge_index[0].astype(jnp.int32))
    col = _prep(edge_index[1].astype(jnp.int32))
    w = _prep(edge_weight.astype(jnp.float32))

    spmm128 = _make_spmm(128)

    t0 = _matmul(x, W0)
    p0 = spmm128(t0, col, row, w)
    t1 = _fused(p0[0], p0[1], b0, W1)
    p1 = spmm128(t1, col, row, w)
    t2 = _combine_relu(p1[0], p1[1], b1)
    p2 = spmm128(t2, col, row, w)
    return _final(p2[0], p2[1], W2, b2)


# speedup vs baseline: 1.5372x; 1.5372x over previous
import functools

import jax
import jax.numpy as jnp
from jax import lax
from jax.experimental import pallas as pl
from jax.experimental.pallas import tpu as pltpu
from jax.experimental.pallas import tpu_sc as plsc

_N = 10000
_E = 320000
_NSC = 2
_NTILE = 16
_NW = _NSC * _NTILE
_RPT = 624
_TAIL = _N - _NTILE * _RPT
_C = 128
_NCH = 79
_EWP = _NCH * _C


def _spmm_body(D, m_hbm, idx_hbm, w_hbm, out_hbm, idxb, wb, rows, acc, sem):
    c = lax.axis_index("c")
    s = lax.axis_index("s")
    wid = c * _NTILE + s

    pltpu.async_copy(idx_hbm.at[wid], idxb, sem)
    pltpu.sync_copy(w_hbm.at[wid], wb)

    def zrow(i, carry):
        for j in range(D // 16):
            rows[i, pl.ds(j * 16, 16)] = jnp.zeros((16,), jnp.float32)
        return carry

    lax.fori_loop(0, _C, zrow, 0)
    base = s * _RPT
    for k in range(_RPT // _C):
        pltpu.sync_copy(rows, acc.at[pl.ds(base + k * _C, _C)])
    _rem = _RPT - (_RPT // _C) * _C
    pltpu.sync_copy(rows.at[pl.ds(0, _rem)],
                    acc.at[pl.ds(base + (_RPT // _C) * _C, _rem)])

    @pl.when(s == _NTILE - 1)
    def _zero_tail():
        pltpu.sync_copy(rows.at[pl.ds(0, _TAIL)],
                        acc.at[pl.ds(_NTILE * _RPT, _TAIL)])

    pltpu.make_async_copy(idx_hbm.at[wid], idxb, sem).wait()
    plsc.subcore_barrier()

    def chunk_body(k, carry):
        pltpu.async_copy(m_hbm.at[idxb.at[k, 0]], rows, sem).wait()

        def scale(g, inner):
            w16 = wb[pl.ds(k * _C + g * 16, 16)]
            for l in range(16):
                e = g * 16 + l
                wspl = jnp.full((16,), w16[l], jnp.float32)
                for j in range(D // 16):
                    rows[e, pl.ds(j * 16, 16)] = (
                        rows[e, pl.ds(j * 16, 16)] * wspl)
            return inner

        lax.fori_loop(0, _C // 16, scale, 0)
        pltpu.sync_copy(rows, acc.at[idxb.at[k, 1]], add=True)
        return carry

    lax.fori_loop(0, _NCH, chunk_body, 0)
    plsc.subcore_barrier()

    pltpu.sync_copy(acc.at[pl.ds(s * _RPT, _RPT)],
                    out_hbm.at[c, pl.ds(s * _RPT, _RPT)])

    @pl.when(s == _NTILE - 1)
    def _write_tail():
        pltpu.sync_copy(acc.at[pl.ds(_NTILE * _RPT, _TAIL)],
                        out_hbm.at[c, pl.ds(_NTILE * _RPT, _TAIL)])


@functools.cache
def _make_spmm(D):
    mesh = plsc.VectorSubcoreMesh(core_axis_name="c", subcore_axis_name="s")
    return pl.kernel(
        functools.partial(_spmm_body, D),
        out_type=jax.ShapeDtypeStruct((_NSC, _N, D), jnp.float32),
        mesh=mesh,
        scratch_types=[
            pltpu.VMEM((_NCH, 2, _C), jnp.int32),
            pltpu.VMEM((_EWP,), jnp.float32),
            pltpu.VMEM((_C, D), jnp.float32),
            pltpu.VMEM_SHARED((_N, D), jnp.float32),
            pltpu.SemaphoreType.DMA,
        ],
        name=f"gcn_spmm_d{D}",
    )


def _matmul_body(x_ref, w_ref, o_ref):
    o_ref[...] = jnp.dot(x_ref[...], w_ref[...],
                         preferred_element_type=jnp.float32)


def _fused_body(p0_ref, p1_ref, b_ref, w_ref, o_ref):
    h = jnp.maximum(p0_ref[...] + p1_ref[...] + b_ref[...], 0.0)
    o_ref[...] = jnp.dot(h, w_ref[...], preferred_element_type=jnp.float32)


def _combine_relu_body(p0_ref, p1_ref, b_ref, o_ref):
    o_ref[...] = jnp.maximum(p0_ref[...] + p1_ref[...] + b_ref[...], 0.0)


def _final_body(p0_ref, p1_ref, w_ref, b_ref, o_ref):
    o_ref[...] = jnp.dot(p0_ref[...] + p1_ref[...], w_ref[...],
                         preferred_element_type=jnp.float32) + b_ref[...]


_BLK = 1000


def _matmul(x, W):
    K, M = W.shape
    return pl.pallas_call(
        _matmul_body,
        grid=(_N // _BLK,),
        in_specs=[
            pl.BlockSpec((_BLK, K), lambda i: (i, 0)),
            pl.BlockSpec((K, M), lambda i: (0, 0)),
        ],
        out_specs=pl.BlockSpec((_BLK, M), lambda i: (i, 0)),
        out_shape=jax.ShapeDtypeStruct((_N, M), jnp.float32),
    )(x, W)


def _fused(p0, p1, b, W):
    K, M = W.shape
    return pl.pallas_call(
        _fused_body,
        grid=(_N // _BLK,),
        in_specs=[
            pl.BlockSpec((_BLK, K), lambda i: (i, 0)),
            pl.BlockSpec((_BLK, K), lambda i: (i, 0)),
            pl.BlockSpec((1, K), lambda i: (0, 0)),
            pl.BlockSpec((K, M), lambda i: (0, 0)),
        ],
        out_specs=pl.BlockSpec((_BLK, M), lambda i: (i, 0)),
        out_shape=jax.ShapeDtypeStruct((_N, M), jnp.float32),
    )(p0, p1, b.reshape(1, K), W)


def _combine_relu(p0, p1, b):
    M = p0.shape[1]
    return pl.pallas_call(
        _combine_relu_body,
        grid=(_N // _BLK,),
        in_specs=[
            pl.BlockSpec((_BLK, M), lambda i: (i, 0)),
            pl.BlockSpec((_BLK, M), lambda i: (i, 0)),
            pl.BlockSpec((1, M), lambda i: (0, 0)),
        ],
        out_specs=pl.BlockSpec((_BLK, M), lambda i: (i, 0)),
        out_shape=jax.ShapeDtypeStruct((_N, M), jnp.float32),
    )(p0, p1, b.reshape(1, M))


def _final(p0, p1, W, b):
    K, M = W.shape
    return pl.pallas_call(
        _final_body,
        grid=(_N // _BLK,),
        in_specs=[
            pl.BlockSpec((_BLK, K), lambda i: (i, 0)),
            pl.BlockSpec((_BLK, K), lambda i: (i, 0)),
            pl.BlockSpec((K, M), lambda i: (0, 0)),
            pl.BlockSpec((1, M), lambda i: (0, 0)),
        ],
        out_specs=pl.BlockSpec((_BLK, M), lambda i: (i, 0)),
        out_shape=jax.ShapeDtypeStruct((_N, M), jnp.float32),
    )(p0, p1, W, b.reshape(1, M))


def kernel(x, edge_index, edge_weight, W0, b0, W1, b1, W2, b2):
    pad = _NW * _EWP - _E

    def _prep(a):
        return jnp.pad(a, (0, pad)).reshape(_NW, _NCH, 1, _C)

    idx = jnp.concatenate(
        [_prep(edge_index[1].astype(jnp.int32)),
         _prep(edge_index[0].astype(jnp.int32))], axis=2)
    w = jnp.pad(edge_weight.astype(jnp.float32),
                (0, pad)).reshape(_NW, _EWP)

    spmm128 = _make_spmm(128)

    t0 = _matmul(x, W0)
    p0 = spmm128(t0, idx, w)
    t1 = _fused(p0[0], p0[1], b0, W1)
    p1 = spmm128(t1, idx, w)
    t2 = _combine_relu(p1[0], p1[1], b1)
    p2 = spmm128(t2, idx, w)
    return _final(p2[0], p2[1], W2, b2)
